# Initial kernel scaffold; baseline (speedup 1.0000x reference)
import jax
import jax.numpy as jnp
from jax.experimental import pallas as pl


def kernel(hidden_states, gate_w, w1, w2, w3, sw1, sw2, sw3):
    raise NotImplementedError("write your pallas kernel here")


# speedup vs baseline: 1.4804x; 1.4804x over previous
import functools

import jax
import jax.numpy as jnp
from jax.experimental import pallas as pl
from jax.experimental.pallas import tpu as pltpu


def _dot_t(a, b):
    return jax.lax.dot_general(a, b, (((1,), (1,)), ((), ())),
                               preferred_element_type=jnp.float32)


def _router_kernel(logits_ref, wmat_ref):
    logits = logits_ref[...]
    t, e = logits.shape
    m = jnp.max(logits, axis=-1, keepdims=True)
    ex = jnp.exp(logits - m)
    scores = ex / jnp.sum(ex, axis=-1, keepdims=True)
    lane = jax.lax.broadcasted_iota(jnp.int32, (t, e), 1)
    s1 = jnp.max(scores, axis=-1, keepdims=True)
    a1 = jnp.min(jnp.where(scores == s1, lane, e), axis=-1, keepdims=True)
    m1 = lane == a1
    scores2 = jnp.where(m1, -1.0, scores)
    s2 = jnp.max(scores2, axis=-1, keepdims=True)
    a2 = jnp.min(jnp.where(scores2 == s2, lane, e), axis=-1, keepdims=True)
    m2 = lane == a2
    denom = s1 + s2 + 1e-6
    wmat_ref[...] = (jnp.where(m1, s1, 0.0) + jnp.where(m2, s2, 0.0)) / denom


def _shared_kernel(x_ref, sw1_ref, sw3_ref, sw2_ref, o_ref):
    x = x_ref[...]
    g = _dot_t(x, sw1_ref[...])
    u = _dot_t(x, sw3_ref[...])
    h = (g * jax.nn.sigmoid(g) * u).astype(jnp.bfloat16)
    o_ref[...] = _dot_t(h, sw2_ref[...])


def _moe_kernel(shared_ref, xb_ref, w1_ref, w3_ref, w2_ref, wmat_ref,
                out_ref, *, bt):
    e = pl.program_id(0)
    t = pl.program_id(1)
    x = xb_ref[...]
    g = _dot_t(x, w1_ref[0])
    u = _dot_t(x, w3_ref[0])
    h = (g * jax.nn.sigmoid(g) * u).astype(jnp.bfloat16)
    y = _dot_t(h, w2_ref[0])
    wm = wmat_ref[...]
    num_e = wm.shape[1]
    sl = pl.ds(t * bt, bt)
    lane = jax.lax.broadcasted_iota(jnp.int32, (bt, num_e), 1)
    wcol = jnp.sum(jnp.where(lane == e, wm, 0.0), axis=-1)
    y = y * wcol[:, None]

    @pl.when(e == 0)
    def _():
        out_ref[sl, :] = shared_ref[sl, :] + y

    @pl.when(e != 0)
    def _():
        out_ref[sl, :] += y


def kernel(hidden_states, gate_w, w1, w2, w3, sw1, sw2, sw3):
    b, s, d = hidden_states.shape
    t = b * s
    e = gate_w.shape[0]
    df = w1.shape[1]
    dsh = sw1.shape[0]
    x = hidden_states.reshape(t, d)

    logits = x @ gate_w.T
    wmat = pl.pallas_call(
        _router_kernel,
        out_shape=jax.ShapeDtypeStruct((t, e), jnp.float32),
    )(logits)

    xb = x.astype(jnp.bfloat16)

    bts = 512
    shared = pl.pallas_call(
        _shared_kernel,
        grid=(t // bts,),
        in_specs=[
            pl.BlockSpec((bts, d), lambda i: (i, 0)),
            pl.BlockSpec((dsh, d), lambda i: (0, 0)),
            pl.BlockSpec((dsh, d), lambda i: (0, 0)),
            pl.BlockSpec((d, dsh), lambda i: (0, 0)),
        ],
        out_specs=pl.BlockSpec((bts, d), lambda i: (i, 0)),
        out_shape=jax.ShapeDtypeStruct((t, d), jnp.float32),
    )(xb, sw1.astype(jnp.bfloat16), sw3.astype(jnp.bfloat16),
      sw2.astype(jnp.bfloat16))

    bt = 512
    out = pl.pallas_call(
        functools.partial(_moe_kernel, bt=bt),
        grid=(e, t // bt),
        in_specs=[
            pl.BlockSpec((t, d), lambda i, j: (0, 0)),
            pl.BlockSpec((bt, d), lambda i, j: (j, 0)),
            pl.BlockSpec((1, df, d), lambda i, j: (i, 0, 0)),
            pl.BlockSpec((1, df, d), lambda i, j: (i, 0, 0)),
            pl.BlockSpec((1, d, df), lambda i, j: (i, 0, 0)),
            pl.BlockSpec((bt, e), lambda i, j: (j, 0)),
        ],
        out_specs=pl.BlockSpec((t, d), lambda i, j: (0, 0)),
        out_shape=jax.ShapeDtypeStruct((t, d), jnp.float32),
        compiler_params=pltpu.CompilerParams(
            dimension_semantics=("arbitrary", "arbitrary")),
    )(shared, xb, w1.astype(jnp.bfloat16), w3.astype(jnp.bfloat16),
      w2.astype(jnp.bfloat16), wmat)

    return out.reshape(b, s, d), logits
